# baseline (speedup 1.0000x reference)
import functools

import jax
import jax.numpy as jnp
from jax import lax
from jax.experimental import pallas as pl
from jax.experimental.pallas import tpu as pltpu
from jax.experimental.pallas import tpu_sc as plsc

T = 2048
H = 768
E = 64
F = 384
TB = 128
C = 32
TP = 2560

NC, NS = 2, 16
NW = NC * NS
BPW = T // NW


def _routing_body(x_ref, g_ref, pos_ref, sid_ref, offs_ref, cnt_ref):
    x = x_ref[...]
    g = g_ref[...]
    logits = lax.dot_general(x, g, (((1,), (1,)), ((), ())),
                             preferred_element_type=jnp.float32)
    m = jnp.max(logits, axis=1, keepdims=True)
    eids = lax.broadcasted_iota(jnp.int32, (T, E), 1)
    e_first = jnp.min(jnp.where(logits == m, eids, E), axis=1, keepdims=True)
    onehot = (eids == e_first).astype(jnp.float32)

    counts = jnp.sum(onehot, axis=0, keepdims=True)
    counts_a = jnp.floor((counts + 7.0) * 0.125) * 8.0
    er = lax.broadcasted_iota(jnp.int32, (E, E), 0)
    ec = lax.broadcasted_iota(jnp.int32, (E, E), 1)
    strict_lower_e = (er < ec).astype(jnp.float32)
    offs = lax.dot_general(counts_a, strict_lower_e, (((1,), (0,)), ((), ())),
                           preferred_element_type=jnp.float32)

    rr = lax.broadcasted_iota(jnp.int32, (TB, TB), 0)
    rc = lax.broadcasted_iota(jnp.int32, (TB, TB), 1)
    strict_lower_t = (rc < rr).astype(jnp.float32)

    base = jnp.zeros((1, E), jnp.float32)
    pos_blocks = []
    for i in range(T // TB):
        ob = onehot[i * TB:(i + 1) * TB, :]
        csb = lax.dot_general(strict_lower_t, ob, (((1,), (0,)), ((), ())),
                              preferred_element_type=jnp.float32)
        posb = jnp.sum((csb + base + offs) * ob, axis=1)
        pos_ref[i, :] = posb.astype(jnp.int32)
        pos_blocks.append(posb[None, :])
        base = base + jnp.sum(ob, axis=0, keepdims=True)

    posi = jnp.concatenate(pos_blocks, axis=1).astype(jnp.int32)
    tids = lax.broadcasted_iota(jnp.int32, (TB, T), 1)
    for i in range(TP // TB):
        pv = i * TB + lax.broadcasted_iota(jnp.int32, (TB, 1), 0)
        hit = posi == pv
        sidb = jnp.sum(jnp.where(hit, tids, 0), axis=1)
        sid_ref[i, :] = sidb

    offs_ref[...] = offs.astype(jnp.int32)
    cnt_ref[...] = counts.astype(jnp.int32)


_routing_call = pl.pallas_call(
    _routing_body,
    out_shape=(
        jax.ShapeDtypeStruct((T // TB, TB), jnp.int32),
        jax.ShapeDtypeStruct((TP // TB, TB), jnp.int32),
        jax.ShapeDtypeStruct((1, E), jnp.int32),
        jax.ShapeDtypeStruct((1, E), jnp.int32),
    ),
)


@functools.cache
def _sc_row_gather_kernel(n_out):
    bpw = n_out // NW

    def body(src_hbm, idx_hbm, out_hbm, idx_v, rows_v, sem):
        wid = lax.axis_index("s") * NC + lax.axis_index("c")
        base = wid * bpw
        pltpu.sync_copy(idx_hbm.at[pl.ds(base, bpw)], idx_v)
        pltpu.async_copy(src_hbm.at[idx_v], rows_v, sem).wait()
        pltpu.sync_copy(rows_v, out_hbm.at[pl.ds(base, bpw)])

    return pl.kernel(
        body,
        mesh=plsc.VectorSubcoreMesh(core_axis_name="c", subcore_axis_name="s"),
        out_type=jax.ShapeDtypeStruct((n_out, H), jnp.float32),
        scratch_types=[
            pltpu.VMEM((bpw,), jnp.int32),
            pltpu.VMEM((bpw, H), jnp.float32),
            pltpu.SemaphoreType.DMA,
        ],
    )


def _sc_row_gather(src, idx):
    return _sc_row_gather_kernel(idx.shape[0])(src, idx)


def _moe_body(offs_ref, cnt_ref, xs_ref, w1_ref, w3_ref, w2_ref, ys_ref):
    e = pl.program_id(0)
    offs = offs_ref[e]
    n = cnt_ref[e]
    w1 = w1_ref[0]
    w3 = w3_ref[0]
    w2 = w2_ref[0]

    def chunk(c, carry):
        start = pl.multiple_of(jnp.minimum(offs + c * C, TP - C), 8)
        xa = xs_ref[pl.ds(start, C), :]
        h1 = lax.dot_general(xa, w1, (((1,), (1,)), ((), ())),
                             preferred_element_type=jnp.float32)
        h3 = lax.dot_general(xa, w3, (((1,), (1,)), ((), ())),
                             preferred_element_type=jnp.float32)
        act = (h1 * lax.logistic(h1)) * h3
        y = lax.dot_general(act, w2, (((1,), (1,)), ((), ())),
                            preferred_element_type=jnp.float32)
        gr = start + lax.broadcasted_iota(jnp.int32, (C, 1), 0)
        mask = (gr >= offs) & (gr < offs + n)
        old = ys_ref[pl.ds(start, C), :]
        ys_ref[pl.ds(start, C), :] = jnp.where(mask, y, old)
        return carry

    nch = (n + C - 1) // C
    lax.fori_loop(0, nch, chunk, 0)


_moe_call = pl.pallas_call(
    _moe_body,
    grid_spec=pltpu.PrefetchScalarGridSpec(
        num_scalar_prefetch=2,
        grid=(E,),
        in_specs=[
            pl.BlockSpec((TP, H), lambda e, o, c: (0, 0)),
            pl.BlockSpec((1, F, H), lambda e, o, c: (e, 0, 0)),
            pl.BlockSpec((1, F, H), lambda e, o, c: (e, 0, 0)),
            pl.BlockSpec((1, H, F), lambda e, o, c: (e, 0, 0)),
        ],
        out_specs=pl.BlockSpec((TP, H), lambda e, o, c: (0, 0)),
    ),
    out_shape=jax.ShapeDtypeStruct((TP, H), jnp.float32),
    compiler_params=pltpu.CompilerParams(
        dimension_semantics=("arbitrary",),
    ),
)


def kernel(hidden_states, gate_weight, w1, w3, w2):
    x = hidden_states.reshape(-1, H)
    pos2d, sid2d, offs2, cnt2 = _routing_call(x, gate_weight)
    pos = pos2d.reshape(T)
    sid = sid2d.reshape(TP)
    offs = offs2.reshape(E)
    cnt = cnt2.reshape(E)
    xs = _sc_row_gather(x, sid)
    ys = _moe_call(offs, cnt, xs, w1, w3, w2)
    out = _sc_row_gather(ys, pos)
    return out


# speedup vs baseline: 1.0778x; 1.0778x over previous
import functools

import jax
import jax.numpy as jnp
from jax import lax
from jax.experimental import pallas as pl
from jax.experimental.pallas import tpu as pltpu
from jax.experimental.pallas import tpu_sc as plsc

T = 2048
H = 768
E = 64
F = 384
TB = 128
C = 128
TP = 2560

NC, NS = 2, 16
NW = NC * NS
BPW = T // NW


def _routing_body(x_ref, g_ref, pos_ref, sid_ref, offs_ref, cnt_ref):
    x = x_ref[...]
    g = g_ref[...]
    logits = lax.dot_general(x, g, (((1,), (1,)), ((), ())),
                             preferred_element_type=jnp.float32)
    m = jnp.max(logits, axis=1, keepdims=True)
    eids = lax.broadcasted_iota(jnp.int32, (T, E), 1)
    e_first = jnp.min(jnp.where(logits == m, eids, E), axis=1, keepdims=True)
    onehot = (eids == e_first).astype(jnp.float32)

    counts = jnp.sum(onehot, axis=0, keepdims=True)
    counts_a = jnp.floor((counts + 7.0) * 0.125) * 8.0
    er = lax.broadcasted_iota(jnp.int32, (E, E), 0)
    ec = lax.broadcasted_iota(jnp.int32, (E, E), 1)
    strict_lower_e = (er < ec).astype(jnp.float32)
    offs = lax.dot_general(counts_a, strict_lower_e, (((1,), (0,)), ((), ())),
                           preferred_element_type=jnp.float32)

    rr = lax.broadcasted_iota(jnp.int32, (TB, TB), 0)
    rc = lax.broadcasted_iota(jnp.int32, (TB, TB), 1)
    strict_lower_t = (rc < rr).astype(jnp.float32)

    base = jnp.zeros((1, E), jnp.float32)
    pos_blocks = []
    for i in range(T // TB):
        ob = onehot[i * TB:(i + 1) * TB, :]
        csb = lax.dot_general(strict_lower_t, ob, (((1,), (0,)), ((), ())),
                              preferred_element_type=jnp.float32)
        posb = jnp.sum((csb + base + offs) * ob, axis=1)
        pos_ref[i, :] = posb.astype(jnp.int32)
        pos_blocks.append(posb[None, :])
        base = base + jnp.sum(ob, axis=0, keepdims=True)

    posi = jnp.concatenate(pos_blocks, axis=1).astype(jnp.int32)
    tids = lax.broadcasted_iota(jnp.int32, (TB, T), 1)
    for i in range(TP // TB):
        pv = i * TB + lax.broadcasted_iota(jnp.int32, (TB, 1), 0)
        hit = posi == pv
        sidb = jnp.sum(jnp.where(hit, tids, 0), axis=1)
        sid_ref[i, :] = sidb

    offs_ref[...] = offs.astype(jnp.int32)
    cnt_ref[...] = counts.astype(jnp.int32)


_routing_call = pl.pallas_call(
    _routing_body,
    out_shape=(
        jax.ShapeDtypeStruct((T // TB, TB), jnp.int32),
        jax.ShapeDtypeStruct((TP // TB, TB), jnp.int32),
        jax.ShapeDtypeStruct((1, E), jnp.int32),
        jax.ShapeDtypeStruct((1, E), jnp.int32),
    ),
)


@functools.cache
def _sc_row_gather_kernel(n_out):
    bpw = n_out // NW

    def body(src_hbm, idx_hbm, out_hbm, idx_v, rows_v, sem):
        wid = lax.axis_index("s") * NC + lax.axis_index("c")
        base = wid * bpw
        pltpu.sync_copy(idx_hbm.at[pl.ds(base, bpw)], idx_v)
        pltpu.async_copy(src_hbm.at[idx_v], rows_v, sem).wait()
        pltpu.sync_copy(rows_v, out_hbm.at[pl.ds(base, bpw)])

    return pl.kernel(
        body,
        mesh=plsc.VectorSubcoreMesh(core_axis_name="c", subcore_axis_name="s"),
        out_type=jax.ShapeDtypeStruct((n_out, H), jnp.float32),
        scratch_types=[
            pltpu.VMEM((bpw,), jnp.int32),
            pltpu.VMEM((bpw, H), jnp.float32),
            pltpu.SemaphoreType.DMA,
        ],
    )


def _sc_row_gather(src, idx):
    return _sc_row_gather_kernel(idx.shape[0])(src, idx)


def _moe_body(offs_ref, cnt_ref, xs_ref, w1_ref, w3_ref, w2_ref, ys_ref):
    e = pl.program_id(0)
    offs = offs_ref[e]
    n = cnt_ref[e]
    w1 = w1_ref[0]
    w3 = w3_ref[0]
    w2 = w2_ref[0]

    def chunk(c, carry):
        start = pl.multiple_of(jnp.minimum(offs + c * C, TP - C), 8)
        xa = xs_ref[pl.ds(start, C), :]
        h1 = lax.dot_general(xa, w1, (((1,), (1,)), ((), ())),
                             preferred_element_type=jnp.float32)
        h3 = lax.dot_general(xa, w3, (((1,), (1,)), ((), ())),
                             preferred_element_type=jnp.float32)
        act = (h1 * lax.logistic(h1)) * h3
        y = lax.dot_general(act, w2, (((1,), (1,)), ((), ())),
                            preferred_element_type=jnp.float32)
        gr = start + lax.broadcasted_iota(jnp.int32, (C, 1), 0)
        mask = (gr >= offs) & (gr < offs + n)
        old = ys_ref[pl.ds(start, C), :]
        ys_ref[pl.ds(start, C), :] = jnp.where(mask, y, old)
        return carry

    nch = (n + C - 1) // C
    lax.fori_loop(0, nch, chunk, 0)


_moe_call = pl.pallas_call(
    _moe_body,
    grid_spec=pltpu.PrefetchScalarGridSpec(
        num_scalar_prefetch=2,
        grid=(E,),
        in_specs=[
            pl.BlockSpec((TP, H), lambda e, o, c: (0, 0)),
            pl.BlockSpec((1, F, H), lambda e, o, c: (e, 0, 0)),
            pl.BlockSpec((1, F, H), lambda e, o, c: (e, 0, 0)),
            pl.BlockSpec((1, H, F), lambda e, o, c: (e, 0, 0)),
        ],
        out_specs=pl.BlockSpec((TP, H), lambda e, o, c: (0, 0)),
    ),
    out_shape=jax.ShapeDtypeStruct((TP, H), jnp.float32),
    compiler_params=pltpu.CompilerParams(
        dimension_semantics=("arbitrary",),
    ),
)


def kernel(hidden_states, gate_weight, w1, w3, w2):
    x = hidden_states.reshape(-1, H)
    pos2d, sid2d, offs2, cnt2 = _routing_call(x, gate_weight)
    pos = pos2d.reshape(T)
    sid = sid2d.reshape(TP)
    offs = offs2.reshape(E)
    cnt = cnt2.reshape(E)
    xs = _sc_row_gather(x, sid)
    ys = _moe_call(offs, cnt, xs, w1, w3, w2)
    out = _sc_row_gather(ys, pos)
    return out


# speedup vs baseline: 1.3111x; 1.2164x over previous
import functools

import jax
import jax.numpy as jnp
from jax import lax
from jax.experimental import pallas as pl
from jax.experimental.pallas import tpu as pltpu
from jax.experimental.pallas import tpu_sc as plsc

T = 2048
H = 768
E = 64
F = 384
TB = 128
C = 64
TP = 2560

NC, NS = 2, 16
NW = NC * NS
BPW = T // NW


def _routing_body(x_ref, g_ref, pos_ref, offs_ref, cnt_ref):
    x = x_ref[...]
    g = g_ref[...]
    logits = lax.dot_general(x, g, (((1,), (1,)), ((), ())),
                             preferred_element_type=jnp.float32)
    m = jnp.max(logits, axis=1, keepdims=True)
    eids = lax.broadcasted_iota(jnp.int32, (T, E), 1)
    e_first = jnp.min(jnp.where(logits == m, eids, E), axis=1, keepdims=True)
    onehot = (eids == e_first).astype(jnp.float32)

    counts = jnp.sum(onehot, axis=0, keepdims=True)
    counts_a = jnp.floor((counts + 7.0) * 0.125) * 8.0
    er = lax.broadcasted_iota(jnp.int32, (E, E), 0)
    ec = lax.broadcasted_iota(jnp.int32, (E, E), 1)
    strict_lower_e = (er < ec).astype(jnp.float32)
    offs = lax.dot_general(counts_a, strict_lower_e, (((1,), (0,)), ((), ())),
                           preferred_element_type=jnp.float32)

    rr = lax.broadcasted_iota(jnp.int32, (TB, TB), 0)
    rc = lax.broadcasted_iota(jnp.int32, (TB, TB), 1)
    strict_lower_t = (rc < rr).astype(jnp.float32)

    base = jnp.zeros((1, E), jnp.float32)
    for i in range(T // TB):
        ob = onehot[i * TB:(i + 1) * TB, :]
        csb = lax.dot_general(strict_lower_t, ob, (((1,), (0,)), ((), ())),
                              preferred_element_type=jnp.float32)
        posb = jnp.sum((csb + base + offs) * ob, axis=1)
        pos_ref[i, :] = posb.astype(jnp.int32)
        base = base + jnp.sum(ob, axis=0, keepdims=True)

    offs_ref[...] = offs.astype(jnp.int32)
    cnt_ref[...] = counts.astype(jnp.int32)


_routing_call = pl.pallas_call(
    _routing_body,
    out_shape=(
        jax.ShapeDtypeStruct((T // TB, TB), jnp.int32),
        jax.ShapeDtypeStruct((1, E), jnp.int32),
        jax.ShapeDtypeStruct((1, E), jnp.int32),
    ),
)


@functools.cache
def _sc_row_gather_kernel(n_out):
    bpw = n_out // NW

    def body(src_hbm, idx_hbm, out_hbm, idx_v, rows_v, sem):
        wid = lax.axis_index("s") * NC + lax.axis_index("c")
        base = wid * bpw
        pltpu.sync_copy(idx_hbm.at[pl.ds(base, bpw)], idx_v)
        pltpu.async_copy(src_hbm.at[idx_v], rows_v, sem).wait()
        pltpu.sync_copy(rows_v, out_hbm.at[pl.ds(base, bpw)])

    return pl.kernel(
        body,
        mesh=plsc.VectorSubcoreMesh(core_axis_name="c", subcore_axis_name="s"),
        out_type=jax.ShapeDtypeStruct((n_out, H), jnp.float32),
        scratch_types=[
            pltpu.VMEM((bpw,), jnp.int32),
            pltpu.VMEM((bpw, H), jnp.float32),
            pltpu.SemaphoreType.DMA,
        ],
    )


def _sc_row_gather(src, idx):
    return _sc_row_gather_kernel(idx.shape[0])(src, idx)


@functools.cache
def _sc_row_scatter_kernel():
    bpw = T // NW

    def body(src_hbm, idx_hbm, out_hbm, idx_v, rows_v, sem):
        wid = lax.axis_index("s") * NC + lax.axis_index("c")
        base = wid * bpw
        pltpu.sync_copy(idx_hbm.at[pl.ds(base, bpw)], idx_v)
        pltpu.sync_copy(src_hbm.at[pl.ds(base, bpw)], rows_v)
        pltpu.async_copy(rows_v, out_hbm.at[idx_v], sem).wait()

    return pl.kernel(
        body,
        mesh=plsc.VectorSubcoreMesh(core_axis_name="c", subcore_axis_name="s"),
        out_type=jax.ShapeDtypeStruct((TP, H), jnp.float32),
        scratch_types=[
            pltpu.VMEM((bpw,), jnp.int32),
            pltpu.VMEM((bpw, H), jnp.float32),
            pltpu.SemaphoreType.DMA,
        ],
    )


def _moe_body(offs_ref, cnt_ref, xs_ref, w1_ref, w3_ref, w2_ref, ys_ref):
    e = pl.program_id(0)
    offs = offs_ref[e]
    n = cnt_ref[e]
    w1 = w1_ref[0]
    w3 = w3_ref[0]
    w2 = w2_ref[0]

    def chunk(c, carry):
        start = pl.multiple_of(jnp.minimum(offs + c * C, TP - C), 8)
        xa = xs_ref[pl.ds(start, C), :]
        h1 = lax.dot_general(xa, w1, (((1,), (1,)), ((), ())),
                             preferred_element_type=jnp.float32)
        h3 = lax.dot_general(xa, w3, (((1,), (1,)), ((), ())),
                             preferred_element_type=jnp.float32)
        act = (h1 * lax.logistic(h1)) * h3
        y = lax.dot_general(act, w2, (((1,), (1,)), ((), ())),
                            preferred_element_type=jnp.float32)
        gr = start + lax.broadcasted_iota(jnp.int32, (C, 1), 0)
        mask = (gr >= offs) & (gr < offs + n)
        old = ys_ref[pl.ds(start, C), :]
        ys_ref[pl.ds(start, C), :] = jnp.where(mask, y, old)
        return carry

    nch = (n + C - 1) // C
    lax.fori_loop(0, nch, chunk, 0)


_moe_call = pl.pallas_call(
    _moe_body,
    grid_spec=pltpu.PrefetchScalarGridSpec(
        num_scalar_prefetch=2,
        grid=(E,),
        in_specs=[
            pl.BlockSpec((TP, H), lambda e, o, c: (0, 0)),
            pl.BlockSpec((1, F, H), lambda e, o, c: (e, 0, 0)),
            pl.BlockSpec((1, F, H), lambda e, o, c: (e, 0, 0)),
            pl.BlockSpec((1, H, F), lambda e, o, c: (e, 0, 0)),
        ],
        out_specs=pl.BlockSpec((TP, H), lambda e, o, c: (0, 0)),
    ),
    out_shape=jax.ShapeDtypeStruct((TP, H), jnp.float32),
    compiler_params=pltpu.CompilerParams(
        dimension_semantics=("arbitrary",),
    ),
)


def kernel(hidden_states, gate_weight, w1, w3, w2):
    x = hidden_states.reshape(-1, H)
    pos2d, offs2, cnt2 = _routing_call(x, gate_weight)
    pos = pos2d.reshape(T)
    offs = offs2.reshape(E)
    cnt = cnt2.reshape(E)
    xs = _sc_row_scatter_kernel()(x, pos)
    ys = _moe_call(offs, cnt, xs, w1, w3, w2)
    out = _sc_row_gather(ys, pos)
    return out


# speedup vs baseline: 1.5519x; 1.1837x over previous
import functools

import jax
import jax.numpy as jnp
from jax import lax
from jax.experimental import pallas as pl
from jax.experimental.pallas import tpu as pltpu
from jax.experimental.pallas import tpu_sc as plsc

T = 2048
H = 768
E = 64
F = 384
TB = 128
C = 64
TP = 2560

NC, NS = 2, 16
NW = NC * NS
BPW = T // NW


def _routing_body(x_ref, g_ref, pos_ref, offs_ref, cnt_ref):
    x = x_ref[...]
    g = g_ref[...]
    logits = lax.dot_general(x, g, (((1,), (1,)), ((), ())),
                             preferred_element_type=jnp.float32)
    m = jnp.max(logits, axis=1, keepdims=True)
    eids = lax.broadcasted_iota(jnp.int32, (T, E), 1)
    e_first = jnp.min(jnp.where(logits == m, eids, E), axis=1, keepdims=True)
    onehot = (eids == e_first).astype(jnp.float32)

    counts = jnp.sum(onehot, axis=0, keepdims=True)
    counts_a = jnp.floor((counts + 7.0) * 0.125) * 8.0
    er = lax.broadcasted_iota(jnp.int32, (E, E), 0)
    ec = lax.broadcasted_iota(jnp.int32, (E, E), 1)
    strict_lower_e = (er < ec).astype(jnp.float32)
    offs = lax.dot_general(counts_a, strict_lower_e, (((1,), (0,)), ((), ())),
                           preferred_element_type=jnp.float32)

    rr = lax.broadcasted_iota(jnp.int32, (TB, TB), 0)
    rc = lax.broadcasted_iota(jnp.int32, (TB, TB), 1)
    strict_lower_t = (rc < rr).astype(jnp.bfloat16)

    base = jnp.zeros((1, E), jnp.float32)
    for i in range(T // TB):
        ob = onehot[i * TB:(i + 1) * TB, :]
        csb = lax.dot_general(strict_lower_t, ob.astype(jnp.bfloat16),
                              (((1,), (0,)), ((), ())),
                              preferred_element_type=jnp.float32)
        posb = jnp.sum((csb + base + offs) * ob, axis=1)
        pos_ref[i, :] = posb.astype(jnp.int32)
        base = base + jnp.sum(ob, axis=0, keepdims=True)

    offs_ref[...] = offs.astype(jnp.int32)
    cnt_ref[...] = counts.astype(jnp.int32)


_routing_call = pl.pallas_call(
    _routing_body,
    out_shape=(
        jax.ShapeDtypeStruct((T // TB, TB), jnp.int32),
        jax.ShapeDtypeStruct((1, E), jnp.int32),
        jax.ShapeDtypeStruct((1, E), jnp.int32),
    ),
)


@functools.cache
def _sc_row_gather_kernel(n_out):
    bpw = n_out // NW

    def body(src_hbm, idx_hbm, out_hbm, idx_v, rows_v, sem):
        wid = lax.axis_index("s") * NC + lax.axis_index("c")
        base = wid * bpw
        pltpu.sync_copy(idx_hbm.at[pl.ds(base, bpw)], idx_v)
        pltpu.async_copy(src_hbm.at[idx_v], rows_v, sem).wait()
        pltpu.sync_copy(rows_v, out_hbm.at[pl.ds(base, bpw)])

    return pl.kernel(
        body,
        mesh=plsc.VectorSubcoreMesh(core_axis_name="c", subcore_axis_name="s"),
        out_type=jax.ShapeDtypeStruct((n_out, H), jnp.float32),
        scratch_types=[
            pltpu.VMEM((bpw,), jnp.int32),
            pltpu.VMEM((bpw, H), jnp.float32),
            pltpu.SemaphoreType.DMA,
        ],
    )


def _sc_row_gather(src, idx):
    return _sc_row_gather_kernel(idx.shape[0])(src, idx)


@functools.cache
def _sc_row_scatter_kernel():
    bpw = T // NW

    def body(src_hbm, idx_hbm, out_hbm, idx_v, rows_v, sem):
        wid = lax.axis_index("s") * NC + lax.axis_index("c")
        base = wid * bpw
        pltpu.sync_copy(idx_hbm.at[pl.ds(base, bpw)], idx_v)
        pltpu.sync_copy(src_hbm.at[pl.ds(base, bpw)], rows_v)
        pltpu.async_copy(rows_v, out_hbm.at[idx_v], sem).wait()

    return pl.kernel(
        body,
        mesh=plsc.VectorSubcoreMesh(core_axis_name="c", subcore_axis_name="s"),
        out_type=jax.ShapeDtypeStruct((TP, H), jnp.float32),
        scratch_types=[
            pltpu.VMEM((bpw,), jnp.int32),
            pltpu.VMEM((bpw, H), jnp.float32),
            pltpu.SemaphoreType.DMA,
        ],
    )


G = 4


def _moe_body(offs_ref, cnt_ref, xs_ref, w1_ref, w3_ref, w2_ref, ys_ref):
  for j in range(G):
    e = pl.program_id(0) * G + j
    offs = offs_ref[e]
    n = cnt_ref[e]
    w13 = jnp.concatenate(
        [w1_ref[j], w3_ref[j]], axis=0).astype(jnp.bfloat16)
    w2 = w2_ref[j].astype(jnp.bfloat16)

    def chunk(c, carry):
        start = pl.multiple_of(jnp.minimum(offs + c * C, TP - C), 8)
        xa = xs_ref[pl.ds(start, C), :].astype(jnp.bfloat16)
        h13 = lax.dot_general(xa, w13, (((1,), (1,)), ((), ())),
                              preferred_element_type=jnp.float32)
        h1 = h13[:, :F]
        h3 = h13[:, F:]
        act = ((h1 * lax.logistic(h1)) * h3).astype(jnp.bfloat16)
        y = lax.dot_general(act, w2, (((1,), (1,)), ((), ())),
                            preferred_element_type=jnp.float32)
        gr = start + lax.broadcasted_iota(jnp.int32, (C, 1), 0)
        mask = (gr >= offs) & (gr < offs + n)
        old = ys_ref[pl.ds(start, C), :]
        ys_ref[pl.ds(start, C), :] = jnp.where(mask, y, old)
        return carry

    nch = (n + C - 1) // C
    lax.fori_loop(0, nch, chunk, 0)


_moe_call = pl.pallas_call(
    _moe_body,
    grid_spec=pltpu.PrefetchScalarGridSpec(
        num_scalar_prefetch=2,
        grid=(E // G,),
        in_specs=[
            pl.BlockSpec((TP, H), lambda e, o, c: (0, 0)),
            pl.BlockSpec((G, F, H), lambda e, o, c: (e, 0, 0)),
            pl.BlockSpec((G, F, H), lambda e, o, c: (e, 0, 0)),
            pl.BlockSpec((G, H, F), lambda e, o, c: (e, 0, 0)),
        ],
        out_specs=pl.BlockSpec((TP, H), lambda e, o, c: (0, 0)),
    ),
    out_shape=jax.ShapeDtypeStruct((TP, H), jnp.float32),
    compiler_params=pltpu.CompilerParams(
        dimension_semantics=("arbitrary",),
    ),
)


def kernel(hidden_states, gate_weight, w1, w3, w2):
    x = hidden_states.reshape(-1, H)
    pos2d, offs2, cnt2 = _routing_call(x, gate_weight)
    pos = pos2d.reshape(T)
    offs = offs2.reshape(E)
    cnt = cnt2.reshape(E)
    xs = _sc_row_scatter_kernel()(x, pos)
    ys = _moe_call(offs, cnt, xs, w1, w3, w2)
    out = _sc_row_gather(ys, pos)
    return out
